# baseline (speedup 1.0000x reference)
import functools

import jax
import jax.numpy as jnp
from jax.experimental import pallas as pl
from jax.experimental.pallas import tpu as pltpu

N = 10000
D = 128
G = 512
TR = 200


def _fused_kernel(adj_ref, x_ref, idx_ref, w0_ref, b0_ref, w1_ref, b1_ref,
                  g1_ref, be1_ref, g2_ref, be2_ref, fw_ref, fb_ref,
                  out_ref, z0_scr, z1_scr, h2_scr):
    p = pl.program_id(0)
    i = pl.program_id(1)

    @pl.when(jnp.logical_and(p == 0, i == 0))
    def _():
        z0_scr[...] = jnp.dot(x_ref[...], w0_ref[...],
                              preferred_element_type=jnp.float32)

    adj = adj_ref[...]

    @pl.when(p == 0)
    def _():
        acc = jnp.dot(adj, z0_scr[...], preferred_element_type=jnp.float32)
        h = jnp.maximum(acc + b0_ref[...], 0.0)
        z1_scr[pl.ds(i * TR, TR), :] = jnp.dot(
            h, w1_ref[...], preferred_element_type=jnp.float32)

    @pl.when(p == 1)
    def _():
        acc = jnp.dot(adj, z1_scr[...], preferred_element_type=jnp.float32)
        h2_scr[pl.ds(i * TR, TR), :] = jnp.maximum(acc + b1_ref[...], 0.0)

    @pl.when(jnp.logical_and(p == 1, i == (N // TR) - 1))
    def _():
        x = h2_scr[...]
        mean1 = jnp.mean(x, axis=0, keepdims=True)
        var1 = jnp.mean((x - mean1) ** 2, axis=0, keepdims=True)
        xn = (x - mean1) / jnp.sqrt(var1 + 1e-5) * g1_ref[...] + be1_ref[...]
        ids = idx_ref[...]
        gi = jax.lax.broadcasted_iota(jnp.int32, (G, N), 0)
        onehot = (gi == ids).astype(jnp.float32)
        pooled = jnp.dot(onehot, xn, preferred_element_type=jnp.float32)
        mean2 = jnp.mean(pooled, axis=0, keepdims=True)
        var2 = jnp.mean((pooled - mean2) ** 2, axis=0, keepdims=True)
        y = (pooled - mean2) / jnp.sqrt(var2 + 1e-5) * g2_ref[...] + be2_ref[...]
        out = jnp.dot(y, fw_ref[...], preferred_element_type=jnp.float32)
        out_ref[...] = jnp.maximum(out + fb_ref[...], 0.0)


def _const(shape):
    return pl.BlockSpec(shape, lambda p, i: tuple(0 for _ in shape))


@functools.partial(jax.jit, static_argnames=("interpret",))
def _run(adj, x, idx, W0, b0, W1, b1, gamma1, beta1, gamma2, beta2,
         fc1_W, fc1_b, interpret=False):
    f32 = jnp.float32
    row = pl.BlockSpec((TR, N), lambda p, i: (i, 0))
    out = pl.pallas_call(
        _fused_kernel,
        grid=(2, N // TR),
        in_specs=[row, _const((N, D)), _const((1, N)), _const((D, D)),
                  _const((1, D)), _const((D, D)), _const((1, D)),
                  _const((1, D)), _const((1, D)), _const((1, D)),
                  _const((1, D)), _const((D, D)), _const((1, D))],
        out_specs=_const((G, D)),
        out_shape=jax.ShapeDtypeStruct((G, D), f32),
        scratch_shapes=[pltpu.VMEM((N, D), f32), pltpu.VMEM((N, D), f32),
                        pltpu.VMEM((N, D), f32)],
        interpret=interpret,
    )(adj, x, idx.reshape(1, N).astype(jnp.int32), W0, b0.reshape(1, D),
      W1, b1.reshape(1, D), gamma1.reshape(1, D), beta1.reshape(1, D),
      gamma2.reshape(1, D), beta2.reshape(1, D), fc1_W, fc1_b.reshape(1, D))
    return out


def kernel(adj, final_features, segment, idx, W0, b0, W1, b1,
           gamma1, beta1, gamma2, beta2, fc1_W, fc1_b):
    return _run(adj, final_features, idx, W0, b0, W1, b1,
                gamma1, beta1, gamma2, beta2, fc1_W, fc1_b)
